# baseline (speedup 1.0000x reference)
import functools

import jax
import jax.numpy as jnp
import numpy as np
from jax import lax
from jax.experimental import pallas as pl
from jax.experimental.pallas import tpu as pltpu
from jax.experimental.pallas import tpu_sc as plsc

N = 10000
E = 320000
D = 128
H = 4
DH = 32
NB = 128
TD = 64
L = 6
MAXR = 30.0
AVG_DEG = 15.57930850982666

NC = 2
NS = 16
NW = NC * NS
BATCH = 128
NBATCH = E // BATCH
NBW = 80
NBATCH_PAD = NW * NBW
NPAD = 10240


def _mesh():
    return plsc.VectorSubcoreMesh(core_axis_name="c", subcore_axis_name="s")


def _wid():
    return lax.axis_index("s") * NC + lax.axis_index("c")


def _span(w):
    b0 = pl.multiple_of(NBW * w, 8)
    nb = jnp.clip(NBATCH - NBW * w, 1, NBW)
    return b0, nb


@functools.partial(jax.jit, static_argnums=(2, 3))
def _sc_gather(table, idx2, dp, tiled):
    K = max(1, min(6, 768 // dp))
    nchunk = (NBW + K - 1) // K

    @functools.partial(
        pl.kernel,
        mesh=_mesh(),
        out_type=jax.ShapeDtypeStruct((E, dp), jnp.float32),
        compiler_params=pltpu.CompilerParams(use_tc_tiling_on_sc=tiled),
        scratch_types=[
            pltpu.VMEM((NBW, BATCH), jnp.int32),
            pltpu.VMEM((K, BATCH, dp), jnp.float32),
            pltpu.SemaphoreType.DMA((K,)),
            pltpu.SemaphoreType.DMA,
        ],
    )
    def k(table_hbm, idx2_hbm, out_hbm, idx_all, rows_v, gsems, ssem):
        w = _wid()
        b0, nb = _span(w)
        pltpu.sync_copy(idx2_hbm.at[pl.ds(b0, NBW)], idx_all)

        def body(c, _):
            js = [jnp.minimum(c * K + i, nb - 1) for i in range(K)]
            outs = [pl.multiple_of((b0 + js[i]) * BATCH, 8) for i in range(K)]
            for i in range(K):
                pltpu.async_copy(table_hbm.at[idx_all.at[js[i]]],
                                 rows_v.at[i], gsems.at[i])
            for i in range(K):
                pltpu.make_async_copy(table_hbm.at[idx_all.at[js[i]]],
                                      rows_v.at[i], gsems.at[i]).wait()
                pltpu.async_copy(rows_v.at[i],
                                 out_hbm.at[pl.ds(outs[i], BATCH)], ssem)
            for i in range(K):
                pltpu.make_async_copy(rows_v.at[i],
                                      out_hbm.at[pl.ds(outs[i], BATCH)],
                                      ssem).wait()
            return 0

        lax.fori_loop(0, nchunk, body, 0)

    return k(table, idx2)


@jax.jit
def _sc_gather_qkv(kvp, q, sidx2, didx2):
    K = 3
    nchunk = (NBW + K - 1) // K

    @functools.partial(
        pl.kernel,
        mesh=_mesh(),
        out_type=[jax.ShapeDtypeStruct((E, D), jnp.float32)] * 2,
        compiler_params=pltpu.CompilerParams(use_tc_tiling_on_sc=True),
        scratch_types=[
            pltpu.VMEM((NBW, BATCH), jnp.int32),
            pltpu.VMEM((NBW, BATCH), jnp.int32),
            pltpu.VMEM((K, BATCH, D), jnp.float32),
            pltpu.VMEM((K, BATCH, D), jnp.float32),
            pltpu.SemaphoreType.DMA((K,)),
            pltpu.SemaphoreType.DMA((K,)),
            pltpu.SemaphoreType.DMA,
        ],
    )
    def k(kv_hbm, q_hbm, sidx2_hbm, didx2_hbm, kv_out_hbm, qg_hbm,
          sidx_all, didx_all, kv_v, q_v, kvsems, qsems, ssem):
        w = _wid()
        b0, nb = _span(w)
        pltpu.sync_copy(sidx2_hbm.at[pl.ds(b0, NBW)], sidx_all)
        pltpu.sync_copy(didx2_hbm.at[pl.ds(b0, NBW)], didx_all)

        def body(c, _):
            js = [jnp.minimum(c * K + i, nb - 1) for i in range(K)]
            outs = [pl.multiple_of((b0 + js[i]) * BATCH, 8) for i in range(K)]
            for i in range(K):
                pltpu.async_copy(kv_hbm.at[sidx_all.at[js[i]]],
                                 kv_v.at[i], kvsems.at[i])
                pltpu.async_copy(q_hbm.at[didx_all.at[js[i]]],
                                 q_v.at[i], qsems.at[i])
            for i in range(K):
                pltpu.make_async_copy(kv_hbm.at[sidx_all.at[js[i]]],
                                      kv_v.at[i], kvsems.at[i]).wait()
                pltpu.async_copy(kv_v.at[i], kv_out_hbm.at[pl.ds(outs[i], BATCH)],
                                 ssem)
                pltpu.make_async_copy(q_hbm.at[didx_all.at[js[i]]],
                                      q_v.at[i], qsems.at[i]).wait()
                pltpu.async_copy(q_v.at[i], qg_hbm.at[pl.ds(outs[i], BATCH)],
                                 ssem)
            for i in range(K):
                pltpu.make_async_copy(kv_v.at[i], kv_out_hbm.at[pl.ds(outs[i], BATCH)],
                                      ssem).wait()
                pltpu.make_async_copy(q_v.at[i], qg_hbm.at[pl.ds(outs[i], BATCH)],
                                      ssem).wait()
            return 0

        lax.fori_loop(0, nchunk, body, 0)

    return k(kvp, q, sidx2, didx2)


@functools.partial(jax.jit, static_argnums=(2, 3, 4))
def _sc_scatter_add(vals, idx2, dp, kq, tiled):
    npad = NPAD if tiled else N
    zeros = jnp.zeros((npad, dp), jnp.float32)
    rps = npad // NS
    nchunk = (NBW + kq - 1) // kq

    @functools.partial(
        pl.kernel,
        mesh=_mesh(),
        out_type=jax.ShapeDtypeStruct((2 * npad, dp), jnp.float32),
        compiler_params=pltpu.CompilerParams(use_tc_tiling_on_sc=tiled),
        scratch_types=[
            pltpu.VMEM((NBW, BATCH), jnp.int32),
            pltpu.VMEM((kq, BATCH, dp), jnp.float32),
            pltpu.SemaphoreType.DMA((kq,)),
            pltpu.VMEM_SHARED((npad, dp), jnp.float32),
        ],
    )
    def k(vals_hbm, idx2_hbm, zeros_hbm, out_hbm, idx_all, vals_v, vsems, acc):
        cid = lax.axis_index("c")
        sid = lax.axis_index("s")
        w = _wid()
        b0, nb = _span(w)
        r0 = pl.multiple_of(sid * rps, 8)
        o0 = pl.multiple_of(cid * npad + sid * rps, 8)
        pltpu.sync_copy(zeros_hbm.at[pl.ds(r0, rps)], acc.at[pl.ds(r0, rps)])
        pltpu.sync_copy(idx2_hbm.at[pl.ds(b0, NBW)], idx_all)
        plsc.subcore_barrier()

        def body(c, _):
            for i in range(kq):
                j = c * kq + i
                v0 = pl.multiple_of((b0 + j) * BATCH, 8)

                @pl.when(j < nb)
                def _():
                    pltpu.async_copy(vals_hbm.at[pl.ds(v0, BATCH)],
                                     vals_v.at[i], vsems.at[i])
            for i in range(kq):
                j = c * kq + i
                v0 = pl.multiple_of((b0 + j) * BATCH, 8)

                @pl.when(j < nb)
                def _():
                    pltpu.make_async_copy(vals_hbm.at[pl.ds(v0, BATCH)],
                                          vals_v.at[i], vsems.at[i]).wait()
                    pltpu.sync_copy(vals_v.at[i], acc.at[idx_all.at[j]], add=True)
            return 0

        lax.fori_loop(0, nchunk, body, 0)
        plsc.subcore_barrier()
        pltpu.sync_copy(acc.at[pl.ds(r0, rps)], out_hbm.at[pl.ds(o0, rps)])

    return k(vals, idx2, zeros).reshape(2, npad, dp)


def _segsum(vals, idx2, dp, kq=2, tiled=True):
    p = _sc_scatter_add(vals, idx2, dp, kq, tiled)
    return p[0, :N] + p[1, :N]


BE = 2560
_WIDTH = MAXR / NB


def _rbf_gate(rsh, w1, w2, w16):
    rcol = rsh[:, 0:1]
    centers = lax.broadcasted_iota(jnp.int32, (1, NB), 1).astype(jnp.float32) * (MAXR / (NB - 1))
    rbf = jnp.exp(-(((rcol - centers) / _WIDTH) ** 2))
    hmid = jax.nn.silu(jnp.dot(rbf, w1, preferred_element_type=jnp.float32))
    gate = jnp.dot(hmid, w2, preferred_element_type=jnp.float32)
    shx = jnp.dot(rsh, w16, preferred_element_type=jnp.float32)
    return gate, shx


def _head_mats():
    d_sx = lax.broadcasted_iota(jnp.int32, (D, 8), 0) // DH
    h_sx = lax.broadcasted_iota(jnp.int32, (D, 8), 1)
    sx = jnp.where(d_sx == h_sx, 1.0 / np.sqrt(DH), 0.0).astype(jnp.float32)
    h_b = lax.broadcasted_iota(jnp.int32, (8, D), 0)
    d_b = lax.broadcasted_iota(jnp.int32, (8, D), 1) // DH
    b8 = jnp.where(h_b == d_b, 1.0, 0.0).astype(jnp.float32)
    return sx, b8


def _attn_body(rshT_b, kv2_b, qg_b, w1_b, w2_b, w16_b, msg_b, exT_b):
    rsh = jnp.transpose(rshT_b[...], (1, 0))
    gate, shg = _rbf_gate(rsh, w1_b[...], w2_b[...], w16_b[...])
    kv2 = kv2_b[...]
    kg = kv2[:, :D].astype(jnp.float32)
    vg = kv2[:, D:].astype(jnp.float32)
    prod = qg_b[...] * kg * gate * shg
    sx, b8 = _head_mats()
    l8 = jnp.dot(prod, sx, preferred_element_type=jnp.float32)
    hcol = lax.broadcasted_iota(jnp.int32, (BE, 8), 1)
    ex8 = jnp.where(hcol < H, jnp.exp(l8), 0.0)
    exb = jnp.dot(ex8, b8, preferred_element_type=jnp.float32)
    msg_b[...] = vg * gate * exb
    exT_b[...] = jnp.transpose(ex8, (1, 0))


@jax.jit
def _tc_edge_attn(rshT, kvg2, qg, w1, w2, w16):
    return pl.pallas_call(
        _attn_body,
        grid=(E // BE,),
        in_specs=[
            pl.BlockSpec((16, BE), lambda b: (0, b)),
            pl.BlockSpec((BE, 2 * D), lambda b: (b, 0)),
            pl.BlockSpec((BE, D), lambda b: (b, 0)),
            pl.BlockSpec((NB, 64), lambda b: (0, 0)),
            pl.BlockSpec((64, D), lambda b: (0, 0)),
            pl.BlockSpec((16, D), lambda b: (0, 0)),
        ],
        out_specs=[
            pl.BlockSpec((BE, D), lambda b: (b, 0)),
            pl.BlockSpec((8, BE), lambda b: (0, b)),
        ],
        out_shape=[jax.ShapeDtypeStruct((E, D), jnp.float32),
                   jax.ShapeDtypeStruct((8, E), jnp.float32)],
    )(rshT, kvg2, qg, w1, w2, w16)


def _deg_body(rshT_b, w1_b, w2_b, w16_b, out_b):
    rsh = jnp.transpose(rshT_b[...], (1, 0))
    wrad, shf = _rbf_gate(rsh, w1_b[...], w2_b[...], w16_b[...])
    out_b[...] = wrad * shf


@jax.jit
def _tc_edge_deg(rshT, w1, w2, w16):
    return pl.pallas_call(
        _deg_body,
        grid=(E // BE,),
        in_specs=[
            pl.BlockSpec((16, BE), lambda b: (0, b)),
            pl.BlockSpec((NB, 64), lambda b: (0, 0)),
            pl.BlockSpec((64, D), lambda b: (0, 0)),
            pl.BlockSpec((16, D), lambda b: (0, 0)),
        ],
        out_specs=pl.BlockSpec((BE, D), lambda b: (b, 0)),
        out_shape=jax.ShapeDtypeStruct((E, D), jnp.float32),
    )(rshT, w1, w2, w16)


def _w16(wsh):
    return jnp.zeros((16, D), jnp.float32).at[1:10].set(wsh)


def _ln(x):
    mu = x.mean(-1, keepdims=True)
    v = ((x - mu) ** 2).mean(-1, keepdims=True)
    return (x - mu) / jnp.sqrt(v + 1e-6)


def kernel(f_in, pos, batch, t, edge_index, params):
    silu = jax.nn.silu
    src2 = jnp.pad(edge_index[0].reshape(NBATCH, BATCH),
                   ((0, NBATCH_PAD - NBATCH), (0, 0)))
    dst2 = jnp.pad(edge_index[1].reshape(NBATCH, BATCH),
                   ((0, NBATCH_PAD - NBATCH), (0, 0)))

    pos_pad = jnp.pad(pos, ((0, 0), (0, 13)))
    ps = _sc_gather(pos_pad, src2, 16, False)
    pd = _sc_gather(pos_pad, dst2, 16, False)
    edge_vec = ps[:, :3] - pd[:, :3]
    r = jnp.sqrt((edge_vec ** 2).sum(-1) + 1e-12)
    u = edge_vec / r[:, None]
    ux, uy, uz = u[:, 0], u[:, 1], u[:, 2]
    s3 = np.sqrt(3.0); s5 = np.sqrt(5.0); s15 = np.sqrt(15.0)
    zr = jnp.zeros_like(r)
    rshT = jnp.stack([r, jnp.ones_like(ux), s3 * ux, s3 * uy, s3 * uz,
                      s15 * ux * uy, s15 * uy * uz,
                      0.5 * s5 * (3.0 * uz * uz - 1.0), s15 * ux * uz,
                      0.5 * s15 * (ux * ux - uy * uy),
                      zr, zr, zr, zr, zr, zr], axis=0)
    half = TD // 2
    freqs = jnp.exp(-np.log(10000.0) * jnp.arange(half) / (half - 1))
    targs = (t * 10000.0)[:, None] * freqs[None, :]
    temb = jnp.concatenate([jnp.sin(targs), jnp.cos(targs)], axis=1)
    atom_emb = params['atom_table'][f_in] + temb @ params['Wt']
    msg_deg = _tc_edge_deg(rshT, params['Wdeg1'], params['Wdeg2'],
                           _w16(params['Wsh_deg']))
    deg = _segsum(msg_deg, dst2, D) / AVG_DEG
    x = atom_emb + deg

    for i in range(L):
        xn = _ln(x)
        q = xn @ params['Wq_%d' % i]
        kv = xn @ jnp.concatenate([params['Wk_%d' % i], params['Wv_%d' % i]], axis=1)
        kvp = lax.bitcast_convert_type(
            kv.astype(jnp.bfloat16).reshape(N, D, 2), jnp.float32)
        kvp_g, qg = _sc_gather_qkv(kvp, q, src2, dst2)
        kvg2 = lax.bitcast_convert_type(kvp_g, jnp.bfloat16).reshape(E, 2 * D)
        msg, exT = _tc_edge_attn(rshT, kvg2, qg, params['Wr1_%d' % i],
                                 params['Wr2_%d' % i], _w16(params['Wsh_%d' % i]))
        num = _segsum(msg, dst2, D)
        ex16 = jnp.pad(exT[:H].T, ((0, 0), (0, 16 - H)))
        den = _segsum(ex16, dst2, 16, kq=6, tiled=False)[:, :H]
        agg = (num.reshape(N, H, DH) / (den[:, :, None] + 1e-9)).reshape(N, D)
        agg = agg @ params['Wo_%d' % i]
        x = x + agg
        x = x + silu(_ln(x) @ params['Wf1_%d' % i]) @ params['Wf2_%d' % i]

    feat = _ln(x @ params['Wfeat'])
    out = silu(feat @ params['Wh1']) @ params['Wh2']
    sigma_min, sigma_max = 0.01, 50.0
    std = sigma_min * (sigma_max / sigma_min) ** t
    return -out / std[:, None]


# speedup vs baseline: 1.8679x; 1.8679x over previous
import functools

import jax
import jax.numpy as jnp
import numpy as np
from jax import lax
from jax.experimental import pallas as pl
from jax.experimental.pallas import tpu as pltpu
from jax.experimental.pallas import tpu_sc as plsc

N = 10000
E = 320000
D = 128
H = 4
DH = 32
NB = 128
TD = 64
L = 6
MAXR = 30.0
AVG_DEG = 15.57930850982666

NC = 2
NS = 16
NW = NC * NS
BATCH = 128
NBATCH = E // BATCH
NBW = 80
NBATCH_PAD = NW * NBW
NPAD = 10240


def _mesh():
    return plsc.VectorSubcoreMesh(core_axis_name="c", subcore_axis_name="s")


def _wid():
    return lax.axis_index("s") * NC + lax.axis_index("c")


def _span(w):
    b0 = pl.multiple_of(NBW * w, 8)
    nb = jnp.clip(NBATCH - NBW * w, 1, NBW)
    return b0, nb


@functools.partial(jax.jit, static_argnums=(2, 3))
def _sc_gather(table, idx2, dp, tiled):
    K = max(1, min(6, 768 // dp))
    nchunk = (NBW + K - 1) // K

    @functools.partial(
        pl.kernel,
        mesh=_mesh(),
        out_type=jax.ShapeDtypeStruct((E, dp), jnp.float32),
        compiler_params=pltpu.CompilerParams(use_tc_tiling_on_sc=tiled),
        scratch_types=[
            pltpu.VMEM((NBW, BATCH), jnp.int32),
            pltpu.VMEM((K, BATCH, dp), jnp.float32),
            pltpu.SemaphoreType.DMA((K,)),
            pltpu.SemaphoreType.DMA,
        ],
    )
    def k(table_hbm, idx2_hbm, out_hbm, idx_all, rows_v, gsems, ssem):
        w = _wid()
        b0, nb = _span(w)
        pltpu.sync_copy(idx2_hbm.at[pl.ds(b0, NBW)], idx_all)

        def body(c, _):
            js = [jnp.minimum(c * K + i, nb - 1) for i in range(K)]
            outs = [pl.multiple_of((b0 + js[i]) * BATCH, 8) for i in range(K)]
            for i in range(K):
                pltpu.async_copy(table_hbm.at[idx_all.at[js[i]]],
                                 rows_v.at[i], gsems.at[i])
            for i in range(K):
                pltpu.make_async_copy(table_hbm.at[idx_all.at[js[i]]],
                                      rows_v.at[i], gsems.at[i]).wait()
                pltpu.async_copy(rows_v.at[i],
                                 out_hbm.at[pl.ds(outs[i], BATCH)], ssem)
            for i in range(K):
                pltpu.make_async_copy(rows_v.at[i],
                                      out_hbm.at[pl.ds(outs[i], BATCH)],
                                      ssem).wait()
            return 0

        lax.fori_loop(0, nchunk, body, 0)

    return k(table, idx2)


@jax.jit
def _sc_gather_qkv(kv, q, sidx2, didx2):
    K = 2
    nchunk = (NBW + K - 1) // K

    @functools.partial(
        pl.kernel,
        mesh=_mesh(),
        out_type=[jax.ShapeDtypeStruct((E, D), jnp.float32)] * 3,
        compiler_params=pltpu.CompilerParams(use_tc_tiling_on_sc=True),
        scratch_types=[
            pltpu.VMEM((NBW, BATCH), jnp.int32),
            pltpu.VMEM((NBW, BATCH), jnp.int32),
            pltpu.VMEM((K, BATCH, 2 * D), jnp.float32),
            pltpu.VMEM((K, BATCH, D), jnp.float32),
            pltpu.SemaphoreType.DMA((K,)),
            pltpu.SemaphoreType.DMA((K,)),
            pltpu.SemaphoreType.DMA,
        ],
    )
    def k(kv_hbm, q_hbm, sidx2_hbm, didx2_hbm, kg_hbm, vg_hbm, qg_hbm,
          sidx_all, didx_all, kv_v, q_v, kvsems, qsems, ssem):
        w = _wid()
        b0, nb = _span(w)
        pltpu.sync_copy(sidx2_hbm.at[pl.ds(b0, NBW)], sidx_all)
        pltpu.sync_copy(didx2_hbm.at[pl.ds(b0, NBW)], didx_all)

        def body(c, _):
            js = [jnp.minimum(c * K + i, nb - 1) for i in range(K)]
            outs = [pl.multiple_of((b0 + js[i]) * BATCH, 8) for i in range(K)]
            for i in range(K):
                pltpu.async_copy(kv_hbm.at[sidx_all.at[js[i]]],
                                 kv_v.at[i], kvsems.at[i])
                pltpu.async_copy(q_hbm.at[didx_all.at[js[i]]],
                                 q_v.at[i], qsems.at[i])
            for i in range(K):
                pltpu.make_async_copy(kv_hbm.at[sidx_all.at[js[i]]],
                                      kv_v.at[i], kvsems.at[i]).wait()
                pltpu.async_copy(kv_v.at[i, :, pl.ds(0, D)],
                                 kg_hbm.at[pl.ds(outs[i], BATCH)], ssem)
                pltpu.async_copy(kv_v.at[i, :, pl.ds(D, D)],
                                 vg_hbm.at[pl.ds(outs[i], BATCH)], ssem)
                pltpu.make_async_copy(q_hbm.at[didx_all.at[js[i]]],
                                      q_v.at[i], qsems.at[i]).wait()
                pltpu.async_copy(q_v.at[i], qg_hbm.at[pl.ds(outs[i], BATCH)],
                                 ssem)
            for i in range(K):
                pltpu.make_async_copy(kv_v.at[i, :, pl.ds(0, D)],
                                      kg_hbm.at[pl.ds(outs[i], BATCH)], ssem).wait()
                pltpu.make_async_copy(kv_v.at[i, :, pl.ds(D, D)],
                                      vg_hbm.at[pl.ds(outs[i], BATCH)], ssem).wait()
                pltpu.make_async_copy(q_v.at[i], qg_hbm.at[pl.ds(outs[i], BATCH)],
                                      ssem).wait()
            return 0

        lax.fori_loop(0, nchunk, body, 0)

    return k(kv, q, sidx2, didx2)


@functools.partial(jax.jit, static_argnums=(2, 3, 4))
def _sc_scatter_add(vals, idx2, dp, kq, tiled):
    npad = NPAD if tiled else N
    zeros = jnp.zeros((npad, dp), jnp.float32)
    rps = npad // NS
    nchunk = (NBW + kq - 1) // kq

    @functools.partial(
        pl.kernel,
        mesh=_mesh(),
        out_type=jax.ShapeDtypeStruct((2 * npad, dp), jnp.float32),
        compiler_params=pltpu.CompilerParams(use_tc_tiling_on_sc=tiled),
        scratch_types=[
            pltpu.VMEM((NBW, BATCH), jnp.int32),
            pltpu.VMEM((kq, BATCH, dp), jnp.float32),
            pltpu.SemaphoreType.DMA((kq,)),
            pltpu.VMEM_SHARED((npad, dp), jnp.float32),
        ],
    )
    def k(vals_hbm, idx2_hbm, zeros_hbm, out_hbm, idx_all, vals_v, vsems, acc):
        cid = lax.axis_index("c")
        sid = lax.axis_index("s")
        w = _wid()
        b0, nb = _span(w)
        r0 = pl.multiple_of(sid * rps, 8)
        o0 = pl.multiple_of(cid * npad + sid * rps, 8)
        pltpu.sync_copy(zeros_hbm.at[pl.ds(r0, rps)], acc.at[pl.ds(r0, rps)])
        pltpu.sync_copy(idx2_hbm.at[pl.ds(b0, NBW)], idx_all)
        plsc.subcore_barrier()

        def body(c, _):
            for i in range(kq):
                j = c * kq + i
                v0 = pl.multiple_of((b0 + j) * BATCH, 8)

                @pl.when(j < nb)
                def _():
                    pltpu.async_copy(vals_hbm.at[pl.ds(v0, BATCH)],
                                     vals_v.at[i], vsems.at[i])
            for i in range(kq):
                j = c * kq + i
                v0 = pl.multiple_of((b0 + j) * BATCH, 8)

                @pl.when(j < nb)
                def _():
                    pltpu.make_async_copy(vals_hbm.at[pl.ds(v0, BATCH)],
                                          vals_v.at[i], vsems.at[i]).wait()
                    pltpu.sync_copy(vals_v.at[i], acc.at[idx_all.at[j]], add=True)
            return 0

        lax.fori_loop(0, nchunk, body, 0)
        plsc.subcore_barrier()
        pltpu.sync_copy(acc.at[pl.ds(r0, rps)], out_hbm.at[pl.ds(o0, rps)])

    return k(vals, idx2, zeros).reshape(2, npad, dp)


def _segsum(vals, idx2, dp, kq=2, tiled=True):
    p = _sc_scatter_add(vals, idx2, dp, kq, tiled)
    return p[0, :N] + p[1, :N]


BE = 2560
_WIDTH = MAXR / NB


def _rbf_gate(rsh, w1, w2, w16):
    rcol = rsh[:, 0:1]
    centers = lax.broadcasted_iota(jnp.int32, (1, NB), 1).astype(jnp.float32) * (MAXR / (NB - 1))
    rbf = jnp.exp(-(((rcol - centers) / _WIDTH) ** 2))
    hmid = jax.nn.silu(jnp.dot(rbf, w1, preferred_element_type=jnp.float32))
    gate = jnp.dot(hmid, w2, preferred_element_type=jnp.float32)
    shx = jnp.dot(rsh, w16, preferred_element_type=jnp.float32)
    return gate, shx


def _head_mats():
    d_sx = lax.broadcasted_iota(jnp.int32, (D, 8), 0) // DH
    h_sx = lax.broadcasted_iota(jnp.int32, (D, 8), 1)
    sx = jnp.where(d_sx == h_sx, 1.0 / np.sqrt(DH), 0.0).astype(jnp.float32)
    h_b = lax.broadcasted_iota(jnp.int32, (8, D), 0)
    d_b = lax.broadcasted_iota(jnp.int32, (8, D), 1) // DH
    b8 = jnp.where(h_b == d_b, 1.0, 0.0).astype(jnp.float32)
    return sx, b8


def _attn_body(rshT_b, kg_b, vg_b, qg_b, w1_b, w2_b, w16_b, msg_b, exT_b):
    rsh = jnp.transpose(rshT_b[...], (1, 0))
    gate, shg = _rbf_gate(rsh, w1_b[...], w2_b[...], w16_b[...])
    prod = qg_b[...] * kg_b[...] * gate * shg
    sx, b8 = _head_mats()
    l8 = jnp.dot(prod, sx, preferred_element_type=jnp.float32)
    hcol = lax.broadcasted_iota(jnp.int32, (BE, 8), 1)
    ex8 = jnp.where(hcol < H, jnp.exp(l8), 0.0)
    exb = jnp.dot(ex8, b8, preferred_element_type=jnp.float32)
    msg_b[...] = vg_b[...] * gate * exb
    exT_b[...] = jnp.transpose(ex8, (1, 0))


@jax.jit
def _tc_edge_attn(rshT, kg, vg, qg, w1, w2, w16):
    return pl.pallas_call(
        _attn_body,
        grid=(E // BE,),
        in_specs=[
            pl.BlockSpec((16, BE), lambda b: (0, b)),
            pl.BlockSpec((BE, D), lambda b: (b, 0)),
            pl.BlockSpec((BE, D), lambda b: (b, 0)),
            pl.BlockSpec((BE, D), lambda b: (b, 0)),
            pl.BlockSpec((NB, 64), lambda b: (0, 0)),
            pl.BlockSpec((64, D), lambda b: (0, 0)),
            pl.BlockSpec((16, D), lambda b: (0, 0)),
        ],
        out_specs=[
            pl.BlockSpec((BE, D), lambda b: (b, 0)),
            pl.BlockSpec((8, BE), lambda b: (0, b)),
        ],
        out_shape=[jax.ShapeDtypeStruct((E, D), jnp.float32),
                   jax.ShapeDtypeStruct((8, E), jnp.float32)],
    )(rshT, kg, vg, qg, w1, w2, w16)


def _deg_body(rshT_b, w1_b, w2_b, w16_b, out_b):
    rsh = jnp.transpose(rshT_b[...], (1, 0))
    wrad, shf = _rbf_gate(rsh, w1_b[...], w2_b[...], w16_b[...])
    out_b[...] = wrad * shf


@jax.jit
def _tc_edge_deg(rshT, w1, w2, w16):
    return pl.pallas_call(
        _deg_body,
        grid=(E // BE,),
        in_specs=[
            pl.BlockSpec((16, BE), lambda b: (0, b)),
            pl.BlockSpec((NB, 64), lambda b: (0, 0)),
            pl.BlockSpec((64, D), lambda b: (0, 0)),
            pl.BlockSpec((16, D), lambda b: (0, 0)),
        ],
        out_specs=pl.BlockSpec((BE, D), lambda b: (b, 0)),
        out_shape=jax.ShapeDtypeStruct((E, D), jnp.float32),
    )(rshT, w1, w2, w16)


def _w16(wsh):
    return jnp.zeros((16, D), jnp.float32).at[1:10].set(wsh)


def _ln(x):
    mu = x.mean(-1, keepdims=True)
    v = ((x - mu) ** 2).mean(-1, keepdims=True)
    return (x - mu) / jnp.sqrt(v + 1e-6)


def kernel(f_in, pos, batch, t, edge_index, params):
    silu = jax.nn.silu
    src2 = jnp.pad(edge_index[0].reshape(NBATCH, BATCH),
                   ((0, NBATCH_PAD - NBATCH), (0, 0)))
    dst2 = jnp.pad(edge_index[1].reshape(NBATCH, BATCH),
                   ((0, NBATCH_PAD - NBATCH), (0, 0)))

    pos_pad = jnp.pad(pos, ((0, 0), (0, 13)))
    ps = _sc_gather(pos_pad, src2, 16, False)
    pd = _sc_gather(pos_pad, dst2, 16, False)
    edge_vec = ps[:, :3] - pd[:, :3]
    r = jnp.sqrt((edge_vec ** 2).sum(-1) + 1e-12)
    u = edge_vec / r[:, None]
    ux, uy, uz = u[:, 0], u[:, 1], u[:, 2]
    s3 = np.sqrt(3.0); s5 = np.sqrt(5.0); s15 = np.sqrt(15.0)
    zr = jnp.zeros_like(r)
    rshT = jnp.stack([r, jnp.ones_like(ux), s3 * ux, s3 * uy, s3 * uz,
                      s15 * ux * uy, s15 * uy * uz,
                      0.5 * s5 * (3.0 * uz * uz - 1.0), s15 * ux * uz,
                      0.5 * s15 * (ux * ux - uy * uy),
                      zr, zr, zr, zr, zr, zr], axis=0)
    half = TD // 2
    freqs = jnp.exp(-np.log(10000.0) * jnp.arange(half) / (half - 1))
    targs = (t * 10000.0)[:, None] * freqs[None, :]
    temb = jnp.concatenate([jnp.sin(targs), jnp.cos(targs)], axis=1)
    atom_emb = params['atom_table'][f_in] + temb @ params['Wt']
    msg_deg = _tc_edge_deg(rshT, params['Wdeg1'], params['Wdeg2'],
                           _w16(params['Wsh_deg']))
    deg = _segsum(msg_deg, dst2, D) / AVG_DEG
    x = atom_emb + deg

    for i in range(L):
        xn = _ln(x)
        q = xn @ params['Wq_%d' % i]
        kv = xn @ jnp.concatenate([params['Wk_%d' % i], params['Wv_%d' % i]], axis=1)
        kg, vg, qg = _sc_gather_qkv(kv, q, src2, dst2)
        msg, exT = _tc_edge_attn(rshT, kg, vg, qg, params['Wr1_%d' % i],
                                 params['Wr2_%d' % i], _w16(params['Wsh_%d' % i]))
        num = _segsum(msg, dst2, D)
        ex16 = jnp.pad(exT[:H].T, ((0, 0), (0, 16 - H)))
        den = _segsum(ex16, dst2, 16, kq=6, tiled=False)[:, :H]
        agg = (num.reshape(N, H, DH) / (den[:, :, None] + 1e-9)).reshape(N, D)
        agg = agg @ params['Wo_%d' % i]
        x = x + agg
        x = x + silu(_ln(x) @ params['Wf1_%d' % i]) @ params['Wf2_%d' % i]

    feat = _ln(x @ params['Wfeat'])
    out = silu(feat @ params['Wh1']) @ params['Wh2']
    sigma_min, sigma_max = 0.01, 50.0
    std = sigma_min * (sigma_max / sigma_min) ** t
    return -out / std[:, None]
